# baseline (speedup 1.0000x reference)
import functools

import jax
import jax.numpy as jnp
from jax import lax
from jax.experimental import pallas as pl
from jax.experimental.pallas import tpu as pltpu
from jax.experimental.pallas import tpu_sc as plsc

_NUM_CORES = 2
_NUM_SUBCORES = 16


def kernel(x, W):
    B, S = x.shape
    _, D = W.shape
    NW = _NUM_CORES * _NUM_SUBCORES
    rows_per_w = S // NW
    CH = 64
    n_ch = rows_per_w // CH

    mesh = plsc.VectorSubcoreMesh(
        core_axis_name="c", subcore_axis_name="s", num_cores=_NUM_CORES
    )

    @functools.partial(
        pl.kernel,
        out_type=jax.ShapeDtypeStruct((B, S, D), jnp.float32),
        mesh=mesh,
        scratch_types=[pltpu.VMEM((CH, D), jnp.float32)],
    )
    def sc_copy(w_hbm, out_hbm, buf):
        wid = lax.axis_index("s") * _NUM_CORES + lax.axis_index("c")
        base0 = wid * rows_per_w
        for k in range(n_ch):
            base = base0 + k * CH
            pltpu.sync_copy(w_hbm.at[pl.ds(base, CH)], buf)
            for b in range(B):
                pltpu.sync_copy(buf, out_hbm.at[b, pl.ds(base, CH)])

    return sc_copy(W)
